# baseline (speedup 1.0000x reference)
import functools

import jax
import jax.numpy as jnp
from jax import lax
from jax.experimental import pallas as pl
from jax.experimental.pallas import tpu as pltpu
from jax.experimental.pallas import tpu_sc as plsc

S, T, A = 32, 91, 64
P, N = 1024, 20
TL = 40
NH = 11
NF = T - NH
NC, NS = 2, 16
L = 16

_f32 = jnp.float32
_i32 = jnp.int32

_SC_IN = (
    ("av", T * A, _i32),
    ("pos", T * A * 2, _f32),
    ("vel", T * A * 2, _f32),
    ("spd", T * A, _f32),
    ("acc", T * A, _f32),
    ("yaw", T * A, _f32),
    ("yrt", T * A, _f32),
    ("typ", A * 3, _i32),
    ("rol", A * 3, _i32),
    ("siz", A * 3, _f32),
    ("tlv", NH * TL, _i32),
    ("tlp", NH * TL * 2, _f32),
    ("tld", NH * TL * 2, _f32),
    ("tls", NH * TL * 5 + 8, _i32),
)

_SC_OUT = (
    ("refpos", A * 2, _f32),
    ("refyaw", A, _f32),
    ("refrot", A * 4, _f32),
    ("scval", A * NH, _i32),
    ("scpos", A * NH * 2, _f32),
    ("scvel", A * NH * 2, _f32),
    ("scspd", A * NH, _f32),
    ("scacc", A * NH, _f32),
    ("scyaw", A * NH, _f32),
    ("scyrt", A * NH, _f32),
    ("gtval", A * NF, _i32),
    ("gtpos", A * NF * 2, _f32),
    ("gtvel", A * NF * 2, _f32),
    ("gtspd", A * NF, _f32),
    ("gtyaw", A * NF, _f32),
    ("otyp", A * 3, _i32),
    ("orol", A * 3, _i32),
    ("osiz", A * 3, _f32),
    ("otlp", NH * TL * 2, _f32),
    ("otld", NH * TL * 2, _f32),
    ("otls", NH * TL * 5 + 8, _i32),
)

_PI = 3.14159265358979
_TWO_PI = 2.0 * _PI
_SIN_C = (1.0, -1.0 / 6, 1.0 / 120, -1.0 / 5040, 1.0 / 362880,
          -1.0 / 39916800, 1.0 / 6227020800)
_COS_C = (1.0, -0.5, 1.0 / 24, -1.0 / 720, 1.0 / 40320, -1.0 / 3628800,
          1.0 / 479001600, -1.0 / 87178291200)


def _horner(x2, coeffs):
    acc = jnp.full(x2.shape, coeffs[-1], _f32)
    for k in reversed(coeffs[:-1]):
        acc = acc * x2 + _f32(k)
    return acc


def _wrap_pi(x):
    t = lax.rem(x + _f32(_PI), _f32(_TWO_PI))
    t = jnp.where((t != 0.0) & (t < 0.0), t + _f32(_TWO_PI), t)
    return t - _f32(_PI)


def _sincos(x):
    r = _wrap_pi(x)
    r2 = r * r
    return r * _horner(r2, _SIN_C), _horner(r2, _COS_C)


def _sc_body(*refs):
    n_in, n_out = len(_SC_IN), len(_SC_OUT)
    in_hbm = refs[:n_in]
    out_hbm = refs[n_in:n_in + n_out]
    in_v = refs[n_in + n_out:n_in + n_out + n_in]
    out_v = refs[n_in + n_out + n_in:n_in + n_out + n_in + n_out]
    sem_in, sem_out = refs[-2], refs[-1]

    w = lax.axis_index("s") * NC + lax.axis_index("c")

    cps = [pltpu.async_copy(h.at[w], v, sem_in) for h, v in zip(in_hbm, in_v)]
    for c in cps:
        c.wait()

    (av_v, pos_v, vel_v, spd_v, acc_v, yaw_v, yrt_v, typ_v, rol_v, siz_v,
     tlv_v, tlp_v, tld_v, tls_v) = in_v
    (orefpos, orefyaw, orefrot, oscval, oscpos, oscvel, oscspd, oscacc,
     oscyaw, oscyrt, ogtval, ogtpos, ogtvel, ogtspd, ogtyaw, otyp, orol,
     osiz, otlp, otld, otls) = out_v

    iota = lax.iota(_i32, L)

    for g in range(A // L):
        a16 = g * L + iota

        lvs = jnp.full((L,), NH - 1, _i32)
        anyv = jnp.zeros((L,), _i32)
        for t in range(NH):
            v = av_v[pl.ds(t * A + g * L, L)]
            lvs = jnp.where(v != 0, jnp.full((L,), t, _i32), lvs)
            anyv = anyv | v

        refyaw = plsc.load_gather(yaw_v, [lvs * A + a16])
        refpx = plsc.load_gather(pos_v, [lvs * (2 * A) + a16 * 2])
        refpy = plsc.load_gather(pos_v, [lvs * (2 * A) + a16 * 2 + 1])
        sn, cs = _sincos(refyaw)

        orefyaw[pl.ds(g * L, L)] = refyaw
        plsc.store_scatter(orefpos, [a16 * 2], refpx)
        plsc.store_scatter(orefpos, [a16 * 2 + 1], refpy)
        plsc.store_scatter(orefrot, [a16 * 4], cs)
        plsc.store_scatter(orefrot, [a16 * 4 + 1], -sn)
        plsc.store_scatter(orefrot, [a16 * 4 + 2], sn)
        plsc.store_scatter(orefrot, [a16 * 4 + 3], cs)

        for t in range(NH):
            vb = av_v[pl.ds(t * A + g * L, L)]
            vf = vb != 0
            plsc.store_scatter(oscval, [a16 * NH + t], vb)
            px = plsc.load_gather(pos_v, [t * (2 * A) + a16 * 2])
            py = plsc.load_gather(pos_v, [t * (2 * A) + a16 * 2 + 1])
            plsc.store_scatter(oscpos, [a16 * (2 * NH) + 2 * t],
                               jnp.where(vf, px, 0.0))
            plsc.store_scatter(oscpos, [a16 * (2 * NH) + 2 * t + 1],
                               jnp.where(vf, py, 0.0))
            vx = plsc.load_gather(vel_v, [t * (2 * A) + a16 * 2])
            vy = plsc.load_gather(vel_v, [t * (2 * A) + a16 * 2 + 1])
            plsc.store_scatter(oscvel, [a16 * (2 * NH) + 2 * t],
                               jnp.where(vf, vx, 0.0))
            plsc.store_scatter(oscvel, [a16 * (2 * NH) + 2 * t + 1],
                               jnp.where(vf, vy, 0.0))
            for src_v, dst_v in ((spd_v, oscspd), (acc_v, oscacc),
                                 (yaw_v, oscyaw), (yrt_v, oscyrt)):
                x = plsc.load_gather(src_v, [t * A + a16])
                plsc.store_scatter(dst_v, [a16 * NH + t],
                                   jnp.where(vf, x, 0.0))

        def gt_body(t, _, a16=a16, refpx=refpx, refpy=refpy, refyaw=refyaw,
                    sn=sn, cs=cs):
            tf = t - NH
            vb = plsc.load_gather(av_v, [t * A + a16])
            vf = vb != 0
            plsc.store_scatter(ogtval, [a16 * NF + tf], vb)
            px = plsc.load_gather(pos_v, [t * (2 * A) + a16 * 2])
            py = plsc.load_gather(pos_v, [t * (2 * A) + a16 * 2 + 1])
            rx = px - refpx
            ry = py - refpy
            plsc.store_scatter(ogtpos, [a16 * (2 * NF) + 2 * tf],
                               jnp.where(vf, rx * cs + ry * sn, 0.0))
            plsc.store_scatter(ogtpos, [a16 * (2 * NF) + 2 * tf + 1],
                               jnp.where(vf, ry * cs - rx * sn, 0.0))
            vx = plsc.load_gather(vel_v, [t * (2 * A) + a16 * 2])
            vy = plsc.load_gather(vel_v, [t * (2 * A) + a16 * 2 + 1])
            plsc.store_scatter(ogtvel, [a16 * (2 * NF) + 2 * tf],
                               jnp.where(vf, vx * cs + vy * sn, 0.0))
            plsc.store_scatter(ogtvel, [a16 * (2 * NF) + 2 * tf + 1],
                               jnp.where(vf, vy * cs - vx * sn, 0.0))
            sp = plsc.load_gather(spd_v, [t * A + a16])
            plsc.store_scatter(ogtspd, [a16 * NF + tf],
                               jnp.where(vf, sp, 0.0))
            yw = plsc.load_gather(yaw_v, [t * A + a16])
            plsc.store_scatter(ogtyaw, [a16 * NF + tf],
                               jnp.where(vf, _wrap_pi(yw - refyaw), 0.0))
            return 0

        lax.fori_loop(NH, T, gt_body, 0)

        af = anyv != 0
        for ch in range(3):
            tv = plsc.load_gather(typ_v, [a16 * 3 + ch])
            plsc.store_scatter(otyp, [a16 * 3 + ch], jnp.where(af, tv, 0))
            rv = plsc.load_gather(rol_v, [a16 * 3 + ch])
            plsc.store_scatter(orol, [a16 * 3 + ch], jnp.where(af, rv, 0))
            sv = plsc.load_gather(siz_v, [a16 * 3 + ch])
            plsc.store_scatter(osiz, [a16 * 3 + ch], jnp.where(af, sv, 0.0))

    tl_max = NH * TL - 1

    def tl2_body(i, _):
        j = i * L + iota
        m = plsc.load_gather(tlv_v, [j // 2])
        mf = m != 0
        xp = plsc.load_gather(tlp_v, [j])
        plsc.store_scatter(otlp, [j], jnp.where(mf, xp, 0.0))
        xd = plsc.load_gather(tld_v, [j])
        plsc.store_scatter(otld, [j], jnp.where(mf, xd, 0.0))
        return 0

    lax.fori_loop(0, (NH * TL * 2) // L, tl2_body, 0)

    def tl5_body(i, _):
        j = i * L + iota
        m = plsc.load_gather(tlv_v, [jnp.minimum(j // 5, tl_max)])
        x = plsc.load_gather(tls_v, [j])
        plsc.store_scatter(otls, [j], jnp.where(m != 0, x, 0))
        return 0

    lax.fori_loop(0, (NH * TL * 5 + 8) // L, tl5_body, 0)

    cps = [pltpu.async_copy(v, h.at[w], sem_out)
           for v, h in zip(out_v, out_hbm)]
    for c in cps:
        c.wait()


_sc_kernel = pl.kernel(
    _sc_body,
    out_type=tuple(jax.ShapeDtypeStruct((S, n), dt) for _, n, dt in _SC_OUT),
    mesh=plsc.VectorSubcoreMesh(core_axis_name="c", subcore_axis_name="s"),
    compiler_params=pltpu.CompilerParams(needs_layout_passes=False),
    scratch_types=(
        tuple(pltpu.VMEM((n,), dt) for _, n, dt in _SC_IN)
        + tuple(pltpu.VMEM((n,), dt) for _, n, dt in _SC_OUT)
        + (pltpu.SemaphoreType.DMA, pltpu.SemaphoreType.DMA)
    ),
)


def _map_body(mv_ref, mvex_ref, mpos_ref, mdir_ref, mtype_ref,
              opos_ref, odir_ref, otype_ref):
    mask = mvex_ref[...].astype(_i32) != 0
    opos_ref[...] = jnp.where(mask, mpos_ref[...], 0.0)
    odir_ref[...] = jnp.where(mask, mdir_ref[...], 0.0)
    anyv = jnp.max(mv_ref[...].astype(_i32), axis=-1, keepdims=True) != 0
    otype_ref[...] = jnp.where(anyv, mtype_ref[...].astype(_i32),
                               0).astype(jnp.int8)


_map_call = pl.pallas_call(
    _map_body,
    grid=(S,),
    in_specs=[
        pl.BlockSpec((1, P, N), lambda i: (i, 0, 0)),
        pl.BlockSpec((1, P, 2 * N), lambda i: (i, 0, 0)),
        pl.BlockSpec((1, P, 2 * N), lambda i: (i, 0, 0)),
        pl.BlockSpec((1, P, 2 * N), lambda i: (i, 0, 0)),
        pl.BlockSpec((1, P, 11), lambda i: (i, 0, 0)),
    ],
    out_specs=[
        pl.BlockSpec((1, P, 2 * N), lambda i: (i, 0, 0)),
        pl.BlockSpec((1, P, 2 * N), lambda i: (i, 0, 0)),
        pl.BlockSpec((1, P, 11), lambda i: (i, 0, 0)),
    ],
    out_shape=[
        jax.ShapeDtypeStruct((S, P, 2 * N), _f32),
        jax.ShapeDtypeStruct((S, P, 2 * N), _f32),
        jax.ShapeDtypeStruct((S, P, 11), jnp.int8),
    ],
)


def kernel(agent_valid, agent_pos, agent_vel, agent_spd, agent_acc,
           agent_yaw_bbox, agent_yaw_rate, agent_type, agent_role,
           agent_size, agent_cmd, map_valid, map_type, map_pos, map_dir,
           tl_valid, tl_state, tl_pos, tl_dir):
    tls_flat = tl_state[:, :NH].astype(_i32).reshape(S, NH * TL * 5)
    tls_pad = jnp.pad(tls_flat, ((0, 0), (0, 8)))
    sc_in = (
        agent_valid.astype(_i32).reshape(S, T * A),
        agent_pos.reshape(S, T * A * 2),
        agent_vel.reshape(S, T * A * 2),
        agent_spd.reshape(S, T * A),
        agent_acc.reshape(S, T * A),
        agent_yaw_bbox.reshape(S, T * A),
        agent_yaw_rate.reshape(S, T * A),
        agent_type.astype(_i32).reshape(S, A * 3),
        agent_role.astype(_i32).reshape(S, A * 3),
        agent_size.reshape(S, A * 3),
        tl_valid[:, :NH].astype(_i32).reshape(S, NH * TL),
        tl_pos[:, :NH].reshape(S, NH * TL * 2),
        tl_dir[:, :NH].reshape(S, NH * TL * 2),
        tls_pad,
    )
    (o_refpos, o_refyaw, o_refrot, o_scval, o_scpos, o_scvel, o_scspd,
     o_scacc, o_scyaw, o_scyrt, o_gtval, o_gtpos, o_gtvel, o_gtspd,
     o_gtyaw, o_typ, o_rol, o_siz, o_tlp, o_tld, o_tls) = _sc_kernel(*sc_in)

    mvex = jnp.repeat(map_valid, 2, axis=-1).astype(jnp.int8)
    o_mpos, o_mdir, o_mtype = _map_call(
        map_valid.astype(jnp.int8),
        mvex,
        map_pos.reshape(S, P, 2 * N),
        map_dir.reshape(S, P, 2 * N),
        map_type.astype(jnp.int8),
    )

    return (
        o_refpos.reshape(S, A, 1, 2),
        o_refyaw.reshape(S, A, 1),
        o_refrot.reshape(S, A, 2, 2),
        agent_type,
        agent_role,
        o_gtval.astype(bool).reshape(S, A, NF),
        o_gtpos.reshape(S, A, NF, 2),
        o_gtspd.reshape(S, A, NF, 1),
        o_gtvel.reshape(S, A, NF, 2),
        o_gtyaw.reshape(S, A, NF, 1),
        agent_cmd,
        o_scval.astype(bool).reshape(S, A, NH),
        o_scpos.reshape(S, A, NH, 2),
        o_scvel.reshape(S, A, NH, 2),
        o_scspd.reshape(S, A, NH, 1),
        o_scacc.reshape(S, A, NH, 1),
        o_scyaw.reshape(S, A, NH, 1),
        o_scyrt.reshape(S, A, NH, 1),
        o_typ.astype(bool).reshape(S, A, 3),
        o_rol.astype(bool).reshape(S, A, 3),
        o_siz.reshape(S, A, 3),
        map_valid,
        o_mtype.astype(bool),
        o_mpos.reshape(S, P, N, 2),
        o_mdir.reshape(S, P, N, 2),
        tl_valid[:, :NH],
        o_tls[:, :NH * TL * 5].astype(bool).reshape(S, NH, TL, 5),
        o_tlp.reshape(S, NH, TL, 2),
        o_tld.reshape(S, NH, TL, 2),
    )


# speedup vs baseline: 1.6909x; 1.6909x over previous
import functools

import jax
import jax.numpy as jnp
from jax import lax
from jax.experimental import pallas as pl
from jax.experimental.pallas import tpu as pltpu
from jax.experimental.pallas import tpu_sc as plsc

S, T, A = 32, 91, 64
P, N = 1024, 20
TL = 40
NH = 11
NF = T - NH
NC, NS = 2, 16
L = 16

_f32 = jnp.float32
_i32 = jnp.int32

_SC_IN = (
    ("av", T * A, _i32),
    ("pos", T * A * 2, _f32),
    ("vel", T * A * 2, _f32),
    ("spd", T * A, _f32),
    ("acc", T * A, _f32),
    ("yaw", T * A, _f32),
    ("yrt", T * A, _f32),
    ("typ", A * 3, _i32),
    ("rol", A * 3, _i32),
    ("siz", A * 3, _f32),
    ("tlv", NH * TL, _i32),
    ("tlp", NH * TL * 2, _f32),
    ("tld", NH * TL * 2, _f32),
    ("tls", NH * TL * 5 + 8, _i32),
)

_SC_OUT = (
    ("refpos", A * 2, _f32),
    ("refyaw", A, _f32),
    ("refrot", A * 4, _f32),
    ("scval", A * NH, _i32),
    ("scpos", A * NH * 2, _f32),
    ("scvel", A * NH * 2, _f32),
    ("scspd", A * NH, _f32),
    ("scacc", A * NH, _f32),
    ("scyaw", A * NH, _f32),
    ("scyrt", A * NH, _f32),
    ("gtval", A * NF, _i32),
    ("gtpos", A * NF * 2, _f32),
    ("gtvel", A * NF * 2, _f32),
    ("gtspd", A * NF, _f32),
    ("gtyaw", A * NF, _f32),
    ("otyp", A * 3, _i32),
    ("orol", A * 3, _i32),
    ("osiz", A * 3, _f32),
    ("otlp", NH * TL * 2, _f32),
    ("otld", NH * TL * 2, _f32),
    ("otls", NH * TL * 5 + 8, _i32),
)

_PI = 3.14159265358979
_TWO_PI = 2.0 * _PI
_SIN_C = (1.0, -1.0 / 6, 1.0 / 120, -1.0 / 5040, 1.0 / 362880,
          -1.0 / 39916800, 1.0 / 6227020800)
_COS_C = (1.0, -0.5, 1.0 / 24, -1.0 / 720, 1.0 / 40320, -1.0 / 3628800,
          1.0 / 479001600, -1.0 / 87178291200)


def _horner(x2, coeffs):
    acc = jnp.full(x2.shape, coeffs[-1], _f32)
    for k in reversed(coeffs[:-1]):
        acc = acc * x2 + _f32(k)
    return acc


def _wrap_pi(x):
    t = lax.rem(x + _f32(_PI), _f32(_TWO_PI))
    t = jnp.where((t != 0.0) & (t < 0.0), t + _f32(_TWO_PI), t)
    return t - _f32(_PI)


def _sincos(x):
    r = _wrap_pi(x)
    r2 = r * r
    return r * _horner(r2, _SIN_C), _horner(r2, _COS_C)


def _sc_body(*refs):
    n_in, n_out = len(_SC_IN), len(_SC_OUT)
    in_hbm = refs[:n_in]
    out_hbm = refs[n_in:n_in + n_out]
    in_v = refs[n_in + n_out:n_in + n_out + n_in]
    out_v = refs[n_in + n_out + n_in:n_in + n_out + n_in + n_out]
    sem_in, sem_out = refs[-2], refs[-1]

    w = lax.axis_index("s") * NC + lax.axis_index("c")

    cps = [pltpu.async_copy(h.at[w], v, sem_in) for h, v in zip(in_hbm, in_v)]
    for c in cps:
        c.wait()

    (av_v, pos_v, vel_v, spd_v, acc_v, yaw_v, yrt_v, typ_v, rol_v, siz_v,
     tlv_v, tlp_v, tld_v, tls_v) = in_v
    (orefpos, orefyaw, orefrot, oscval, oscpos, oscvel, oscspd, oscacc,
     oscyaw, oscyrt, ogtval, ogtpos, ogtvel, ogtspd, ogtyaw, otyp, orol,
     osiz, otlp, otld, otls) = out_v

    iota = lax.iota(_i32, L)

    for g in range(A // L):
        a16 = g * L + iota

        lvs = jnp.full((L,), NH - 1, _i32)
        anyv = jnp.zeros((L,), _i32)
        for t in range(NH):
            v = av_v[pl.ds(t * A + g * L, L)]
            lvs = jnp.where(v != 0, jnp.full((L,), t, _i32), lvs)
            anyv = anyv | v

        refyaw = plsc.load_gather(yaw_v, [lvs * A + a16])
        refpx = plsc.load_gather(pos_v, [lvs * (2 * A) + a16 * 2])
        refpy = plsc.load_gather(pos_v, [lvs * (2 * A) + a16 * 2 + 1])
        sn, cs = _sincos(refyaw)

        orefyaw[pl.ds(g * L, L)] = refyaw
        plsc.store_scatter(orefpos, [a16 * 2], refpx)
        plsc.store_scatter(orefpos, [a16 * 2 + 1], refpy)
        plsc.store_scatter(orefrot, [a16 * 4], cs)
        plsc.store_scatter(orefrot, [a16 * 4 + 1], -sn)
        plsc.store_scatter(orefrot, [a16 * 4 + 2], sn)
        plsc.store_scatter(orefrot, [a16 * 4 + 3], cs)

        for t in range(NH):
            vb = av_v[pl.ds(t * A + g * L, L)]
            vf = vb != 0
            plsc.store_scatter(oscval, [a16 * NH + t], vb)
            px = plsc.load_gather(pos_v, [t * (2 * A) + a16 * 2])
            py = plsc.load_gather(pos_v, [t * (2 * A) + a16 * 2 + 1])
            plsc.store_scatter(oscpos, [a16 * (2 * NH) + 2 * t],
                               jnp.where(vf, px, 0.0))
            plsc.store_scatter(oscpos, [a16 * (2 * NH) + 2 * t + 1],
                               jnp.where(vf, py, 0.0))
            vx = plsc.load_gather(vel_v, [t * (2 * A) + a16 * 2])
            vy = plsc.load_gather(vel_v, [t * (2 * A) + a16 * 2 + 1])
            plsc.store_scatter(oscvel, [a16 * (2 * NH) + 2 * t],
                               jnp.where(vf, vx, 0.0))
            plsc.store_scatter(oscvel, [a16 * (2 * NH) + 2 * t + 1],
                               jnp.where(vf, vy, 0.0))
            for src_v, dst_v in ((spd_v, oscspd), (acc_v, oscacc),
                                 (yaw_v, oscyaw), (yrt_v, oscyrt)):
                x = plsc.load_gather(src_v, [t * A + a16])
                plsc.store_scatter(dst_v, [a16 * NH + t],
                                   jnp.where(vf, x, 0.0))

        def gt_body(t, _, a16=a16, refpx=refpx, refpy=refpy, refyaw=refyaw,
                    sn=sn, cs=cs):
            tf = t - NH
            vb = plsc.load_gather(av_v, [t * A + a16])
            vf = vb != 0
            plsc.store_scatter(ogtval, [a16 * NF + tf], vb)
            px = plsc.load_gather(pos_v, [t * (2 * A) + a16 * 2])
            py = plsc.load_gather(pos_v, [t * (2 * A) + a16 * 2 + 1])
            rx = px - refpx
            ry = py - refpy
            plsc.store_scatter(ogtpos, [a16 * (2 * NF) + 2 * tf],
                               jnp.where(vf, rx * cs + ry * sn, 0.0))
            plsc.store_scatter(ogtpos, [a16 * (2 * NF) + 2 * tf + 1],
                               jnp.where(vf, ry * cs - rx * sn, 0.0))
            vx = plsc.load_gather(vel_v, [t * (2 * A) + a16 * 2])
            vy = plsc.load_gather(vel_v, [t * (2 * A) + a16 * 2 + 1])
            plsc.store_scatter(ogtvel, [a16 * (2 * NF) + 2 * tf],
                               jnp.where(vf, vx * cs + vy * sn, 0.0))
            plsc.store_scatter(ogtvel, [a16 * (2 * NF) + 2 * tf + 1],
                               jnp.where(vf, vy * cs - vx * sn, 0.0))
            sp = plsc.load_gather(spd_v, [t * A + a16])
            plsc.store_scatter(ogtspd, [a16 * NF + tf],
                               jnp.where(vf, sp, 0.0))
            yw = plsc.load_gather(yaw_v, [t * A + a16])
            plsc.store_scatter(ogtyaw, [a16 * NF + tf],
                               jnp.where(vf, _wrap_pi(yw - refyaw), 0.0))
            return 0

        lax.fori_loop(NH, T, gt_body, 0)

        af = anyv != 0
        for ch in range(3):
            tv = plsc.load_gather(typ_v, [a16 * 3 + ch])
            plsc.store_scatter(otyp, [a16 * 3 + ch], jnp.where(af, tv, 0))
            rv = plsc.load_gather(rol_v, [a16 * 3 + ch])
            plsc.store_scatter(orol, [a16 * 3 + ch], jnp.where(af, rv, 0))
            sv = plsc.load_gather(siz_v, [a16 * 3 + ch])
            plsc.store_scatter(osiz, [a16 * 3 + ch], jnp.where(af, sv, 0.0))

    tl_max = NH * TL - 1

    def tl2_body(i, _):
        j = i * L + iota
        m = plsc.load_gather(tlv_v, [j // 2])
        mf = m != 0
        xp = plsc.load_gather(tlp_v, [j])
        plsc.store_scatter(otlp, [j], jnp.where(mf, xp, 0.0))
        xd = plsc.load_gather(tld_v, [j])
        plsc.store_scatter(otld, [j], jnp.where(mf, xd, 0.0))
        return 0

    lax.fori_loop(0, (NH * TL * 2) // L, tl2_body, 0)

    def tl5_body(i, _):
        j = i * L + iota
        m = plsc.load_gather(tlv_v, [jnp.minimum(j // 5, tl_max)])
        x = plsc.load_gather(tls_v, [j])
        plsc.store_scatter(otls, [j], jnp.where(m != 0, x, 0))
        return 0

    lax.fori_loop(0, (NH * TL * 5 + 8) // L, tl5_body, 0)

    cps = [pltpu.async_copy(v, h.at[w], sem_out)
           for v, h in zip(out_v, out_hbm)]
    for c in cps:
        c.wait()


_sc_kernel = pl.kernel(
    _sc_body,
    out_type=tuple(jax.ShapeDtypeStruct((S, n), dt) for _, n, dt in _SC_OUT),
    mesh=plsc.VectorSubcoreMesh(core_axis_name="c", subcore_axis_name="s"),
    compiler_params=pltpu.CompilerParams(needs_layout_passes=False),
    scratch_types=(
        tuple(pltpu.VMEM((n,), dt) for _, n, dt in _SC_IN)
        + tuple(pltpu.VMEM((n,), dt) for _, n, dt in _SC_OUT)
        + (pltpu.SemaphoreType.DMA, pltpu.SemaphoreType.DMA)
    ),
)


_PB = 256


def _map_body(mv_ref, mt_ref, mpos_ref, mdir_ref,
              opos_ref, odir_ref, otype_ref):
    mv = mv_ref[...].astype(_i32)
    for n in range(N):
        m = (mv[n] != 0)[:, None, :]
        opos_ref[:, n] = jnp.where(m, mpos_ref[:, n], 0.0)
        odir_ref[:, n] = jnp.where(m, mdir_ref[:, n], 0.0)
    anyv = jnp.max(mv, axis=0, keepdims=True) != 0
    otype_ref[...] = jnp.where(anyv, mt_ref[...].astype(_i32),
                               0).astype(jnp.int8)


_map_call = pl.pallas_call(
    _map_body,
    grid=(P // _PB,),
    in_specs=[
        pl.BlockSpec((N, S, _PB), lambda i: (0, 0, i)),
        pl.BlockSpec((11, S, _PB), lambda i: (0, 0, i)),
        pl.BlockSpec((S, N, 2, _PB), lambda i: (0, 0, 0, i)),
        pl.BlockSpec((S, N, 2, _PB), lambda i: (0, 0, 0, i)),
    ],
    out_specs=[
        pl.BlockSpec((S, N, 2, _PB), lambda i: (0, 0, 0, i)),
        pl.BlockSpec((S, N, 2, _PB), lambda i: (0, 0, 0, i)),
        pl.BlockSpec((11, S, _PB), lambda i: (0, 0, i)),
    ],
    out_shape=[
        jax.ShapeDtypeStruct((S, N, 2, P), _f32),
        jax.ShapeDtypeStruct((S, N, 2, P), _f32),
        jax.ShapeDtypeStruct((11, S, P), jnp.int8),
    ],
)


def kernel(agent_valid, agent_pos, agent_vel, agent_spd, agent_acc,
           agent_yaw_bbox, agent_yaw_rate, agent_type, agent_role,
           agent_size, agent_cmd, map_valid, map_type, map_pos, map_dir,
           tl_valid, tl_state, tl_pos, tl_dir):
    tls_flat = tl_state[:, :NH].astype(_i32).reshape(S, NH * TL * 5)
    tls_pad = jnp.pad(tls_flat, ((0, 0), (0, 8)))
    sc_in = (
        agent_valid.astype(_i32).reshape(S, T * A),
        agent_pos.reshape(S, T * A * 2),
        agent_vel.reshape(S, T * A * 2),
        agent_spd.reshape(S, T * A),
        agent_acc.reshape(S, T * A),
        agent_yaw_bbox.reshape(S, T * A),
        agent_yaw_rate.reshape(S, T * A),
        agent_type.astype(_i32).reshape(S, A * 3),
        agent_role.astype(_i32).reshape(S, A * 3),
        agent_size.reshape(S, A * 3),
        tl_valid[:, :NH].astype(_i32).reshape(S, NH * TL),
        tl_pos[:, :NH].reshape(S, NH * TL * 2),
        tl_dir[:, :NH].reshape(S, NH * TL * 2),
        tls_pad,
    )
    (o_refpos, o_refyaw, o_refrot, o_scval, o_scpos, o_scvel, o_scspd,
     o_scacc, o_scyaw, o_scyrt, o_gtval, o_gtpos, o_gtvel, o_gtspd,
     o_gtyaw, o_typ, o_rol, o_siz, o_tlp, o_tld, o_tls) = _sc_kernel(*sc_in)

    o_mpos, o_mdir, o_mtype = _map_call(
        map_valid.transpose(2, 0, 1).astype(jnp.int8),
        map_type.transpose(2, 0, 1).astype(jnp.int8),
        map_pos.transpose(0, 2, 3, 1),
        map_dir.transpose(0, 2, 3, 1),
    )

    return (
        o_refpos.reshape(S, A, 1, 2),
        o_refyaw.reshape(S, A, 1),
        o_refrot.reshape(S, A, 2, 2),
        agent_type,
        agent_role,
        o_gtval.astype(bool).reshape(S, A, NF),
        o_gtpos.reshape(S, A, NF, 2),
        o_gtspd.reshape(S, A, NF, 1),
        o_gtvel.reshape(S, A, NF, 2),
        o_gtyaw.reshape(S, A, NF, 1),
        agent_cmd,
        o_scval.astype(bool).reshape(S, A, NH),
        o_scpos.reshape(S, A, NH, 2),
        o_scvel.reshape(S, A, NH, 2),
        o_scspd.reshape(S, A, NH, 1),
        o_scacc.reshape(S, A, NH, 1),
        o_scyaw.reshape(S, A, NH, 1),
        o_scyrt.reshape(S, A, NH, 1),
        o_typ.astype(bool).reshape(S, A, 3),
        o_rol.astype(bool).reshape(S, A, 3),
        o_siz.reshape(S, A, 3),
        map_valid,
        o_mtype.transpose(1, 2, 0).astype(bool),
        o_mpos.transpose(0, 3, 1, 2),
        o_mdir.transpose(0, 3, 1, 2),
        tl_valid[:, :NH],
        o_tls[:, :NH * TL * 5].astype(bool).reshape(S, NH, TL, 5),
        o_tlp.reshape(S, NH, TL, 2),
        o_tld.reshape(S, NH, TL, 2),
    )


# speedup vs baseline: 1.7889x; 1.0579x over previous
import jax
import jax.numpy as jnp
from jax import lax
from jax.experimental import pallas as pl
from jax.experimental.pallas import tpu as pltpu
from jax.experimental.pallas import tpu_sc as plsc

S, T, A = 32, 91, 64
P, N = 1024, 20
TL = 40
NH = 11
NF = T - NH
NC, NS = 2, 16
L = 16

_f32 = jnp.float32
_i32 = jnp.int32

_SC_IN = (
    ("av", A * T, _i32),
    ("pos", A * 2 * T, _f32),
    ("vel", A * 2 * T, _f32),
    ("spd", A * T, _f32),
    ("acc", A * T, _f32),
    ("yaw", A * T, _f32),
    ("yrt", A * T, _f32),
    ("typ", A * 3, _i32),
    ("rol", A * 3, _i32),
    ("siz", A * 3, _f32),
    ("tlv", TL * T, _i32),
    ("tlp", TL * 2 * T, _f32),
    ("tld", TL * 2 * T, _f32),
)

_SC_OUT = (
    ("refpos", A * 2, _f32),
    ("refyaw", A, _f32),
    ("refrot", A * 4, _f32),
    ("scval", A * NH, _i32),
    ("scpos", A * NH * 2, _f32),
    ("scvel", A * NH * 2, _f32),
    ("scspd", A * NH, _f32),
    ("scacc", A * NH, _f32),
    ("scyaw", A * NH, _f32),
    ("scyrt", A * NH, _f32),
    ("gtval", A * NF, _i32),
    ("gtpos", A * NF * 2, _f32),
    ("gtvel", A * NF * 2, _f32),
    ("gtspd", A * NF, _f32),
    ("gtyaw", A * NF, _f32),
    ("otyp", A * 3, _i32),
    ("orol", A * 3, _i32),
    ("osiz", A * 3, _f32),
    ("otlp", NH * TL * 2, _f32),
    ("otld", NH * TL * 2, _f32),
    ("otls", NH * TL * 5 + 8, _i32),
)

_PI = 3.14159265358979
_TWO_PI = 2.0 * _PI
_SIN_C = (1.0, -1.0 / 6, 1.0 / 120, -1.0 / 5040, 1.0 / 362880,
          -1.0 / 39916800, 1.0 / 6227020800)
_COS_C = (1.0, -0.5, 1.0 / 24, -1.0 / 720, 1.0 / 40320, -1.0 / 3628800,
          1.0 / 479001600, -1.0 / 87178291200)


def _horner(x2, coeffs):
    acc = jnp.full(x2.shape, coeffs[-1], _f32)
    for k in reversed(coeffs[:-1]):
        acc = acc * x2 + _f32(k)
    return acc


def _wrap_pi(x):
    t = lax.rem(x + _f32(_PI), _f32(_TWO_PI))
    t = jnp.where((t != 0.0) & (t < 0.0), t + _f32(_TWO_PI), t)
    return t - _f32(_PI)


def _sincos(x):
    r = _wrap_pi(x)
    r2 = r * r
    return r * _horner(r2, _SIN_C), _horner(r2, _COS_C)


def _sc_body(*refs):
    n_in, n_out = len(_SC_IN) + 1, len(_SC_OUT)
    in_hbm = refs[:n_in]
    out_hbm = refs[n_in:n_in + n_out]
    in_v = refs[n_in + n_out:n_in + n_out + n_in]
    out_v = refs[n_in + n_out + n_in:n_in + n_out + n_in + n_out]
    sem_in, sem_out = refs[-2], refs[-1]

    w = lax.axis_index("s") * NC + lax.axis_index("c")

    tls_h = in_hbm[-1]
    cps = [pltpu.async_copy(h.at[w], v, sem_in)
           for h, v in zip(in_hbm[:-1], in_v[:-1])]
    for c in cps:
        c.wait()

    (av_v, pos_v, vel_v, spd_v, acc_v, yaw_v, yrt_v, typ_v, rol_v, siz_v,
     tlv_v, tlp_v, tld_v, tls_v) = in_v
    (orefpos, orefyaw, orefrot, oscval, oscpos, oscvel, oscspd, oscacc,
     oscyaw, oscyrt, ogtval, ogtpos, ogtvel, ogtspd, ogtyaw, otyp, orol,
     osiz, otlp, otld, otls) = out_v

    iota = lax.iota(_i32, L)

    for g in range(A // L):
        a16 = g * L + iota
        aT = a16 * T
        a2T = a16 * (2 * T)

        lvs = jnp.full((L,), NH - 1, _i32)
        anyv = jnp.zeros((L,), _i32)
        for t in range(NH):
            v = plsc.load_gather(av_v, [aT + t])
            lvs = jnp.where(v != 0, jnp.full((L,), t, _i32), lvs)
            anyv = anyv | v

        refyaw = plsc.load_gather(yaw_v, [aT + lvs])
        refpx = plsc.load_gather(pos_v, [a2T + lvs])
        refpy = plsc.load_gather(pos_v, [a2T + T + lvs])
        sn, cs = _sincos(refyaw)

        orefyaw[pl.ds(g * L, L)] = refyaw
        plsc.store_scatter(orefpos, [a16 * 2], refpx)
        plsc.store_scatter(orefpos, [a16 * 2 + 1], refpy)
        plsc.store_scatter(orefrot, [a16 * 4], cs)
        plsc.store_scatter(orefrot, [a16 * 4 + 1], -sn)
        plsc.store_scatter(orefrot, [a16 * 4 + 2], sn)
        plsc.store_scatter(orefrot, [a16 * 4 + 3], cs)

        for t in range(NH):
            vb = plsc.load_gather(av_v, [aT + t])
            vf = vb != 0
            plsc.store_scatter(oscval, [a16 * NH + t], vb)
            px = plsc.load_gather(pos_v, [a2T + t])
            py = plsc.load_gather(pos_v, [a2T + T + t])
            plsc.store_scatter(oscpos, [a16 * (2 * NH) + 2 * t],
                               jnp.where(vf, px, 0.0))
            plsc.store_scatter(oscpos, [a16 * (2 * NH) + 2 * t + 1],
                               jnp.where(vf, py, 0.0))
            vx = plsc.load_gather(vel_v, [a2T + t])
            vy = plsc.load_gather(vel_v, [a2T + T + t])
            plsc.store_scatter(oscvel, [a16 * (2 * NH) + 2 * t],
                               jnp.where(vf, vx, 0.0))
            plsc.store_scatter(oscvel, [a16 * (2 * NH) + 2 * t + 1],
                               jnp.where(vf, vy, 0.0))
            for src_v, dst_v in ((spd_v, oscspd), (acc_v, oscacc),
                                 (yaw_v, oscyaw), (yrt_v, oscyrt)):
                x = plsc.load_gather(src_v, [aT + t])
                plsc.store_scatter(dst_v, [a16 * NH + t],
                                   jnp.where(vf, x, 0.0))

        def gt_body(t, _, a16=a16, aT=aT, a2T=a2T, refpx=refpx, refpy=refpy,
                    refyaw=refyaw, sn=sn, cs=cs):
            tf = t - NH
            vb = plsc.load_gather(av_v, [aT + t])
            vf = vb != 0
            plsc.store_scatter(ogtval, [a16 * NF + tf], vb)
            px = plsc.load_gather(pos_v, [a2T + t])
            py = plsc.load_gather(pos_v, [a2T + T + t])
            rx = px - refpx
            ry = py - refpy
            plsc.store_scatter(ogtpos, [a16 * (2 * NF) + 2 * tf],
                               jnp.where(vf, rx * cs + ry * sn, 0.0))
            plsc.store_scatter(ogtpos, [a16 * (2 * NF) + 2 * tf + 1],
                               jnp.where(vf, ry * cs - rx * sn, 0.0))
            vx = plsc.load_gather(vel_v, [a2T + t])
            vy = plsc.load_gather(vel_v, [a2T + T + t])
            plsc.store_scatter(ogtvel, [a16 * (2 * NF) + 2 * tf],
                               jnp.where(vf, vx * cs + vy * sn, 0.0))
            plsc.store_scatter(ogtvel, [a16 * (2 * NF) + 2 * tf + 1],
                               jnp.where(vf, vy * cs - vx * sn, 0.0))
            sp = plsc.load_gather(spd_v, [aT + t])
            plsc.store_scatter(ogtspd, [a16 * NF + tf],
                               jnp.where(vf, sp, 0.0))
            yw = plsc.load_gather(yaw_v, [aT + t])
            plsc.store_scatter(ogtyaw, [a16 * NF + tf],
                               jnp.where(vf, _wrap_pi(yw - refyaw), 0.0))
            return 0

        lax.fori_loop(NH, T, gt_body, 0)

        af = anyv != 0
        for ch in range(3):
            tv_ = plsc.load_gather(typ_v, [a16 * 3 + ch])
            plsc.store_scatter(otyp, [a16 * 3 + ch], jnp.where(af, tv_, 0))
            rv = plsc.load_gather(rol_v, [a16 * 3 + ch])
            plsc.store_scatter(orol, [a16 * 3 + ch], jnp.where(af, rv, 0))
            sv = plsc.load_gather(siz_v, [a16 * 3 + ch])
            plsc.store_scatter(osiz, [a16 * 3 + ch], jnp.where(af, sv, 0.0))

    def tl2_body(i, _):
        j = i * L + iota
        t = j // (2 * TL)
        r = j - t * (2 * TL)
        l = r // 2
        c = r - l * 2
        m = plsc.load_gather(tlv_v, [l * T + t])
        mf = m != 0
        xp = plsc.load_gather(tlp_v, [l * (2 * T) + c * T + t])
        plsc.store_scatter(otlp, [j], jnp.where(mf, xp, 0.0))
        xd = plsc.load_gather(tld_v, [l * (2 * T) + c * T + t])
        plsc.store_scatter(otld, [j], jnp.where(mf, xd, 0.0))
        return 0

    lax.fori_loop(0, (NH * TL * 2) // L, tl2_body, 0)

    for q in range(5):
        pltpu.sync_copy(tls_h.at[w, q], tls_v)

        def tl5_body(i, _, q=q):
            k = i * L + iota
            t = k // TL
            l = k - t * TL
            m = plsc.load_gather(tlv_v, [l * T + t])
            x = plsc.load_gather(tls_v, [l, jnp.minimum(t, NH - 1)])
            jj = t * (TL * 5) + l * 5 + q
            plsc.store_scatter(otls, [jj], jnp.where(m != 0, x, 0),
                               mask=k < NH * TL)
            return 0

        lax.fori_loop(0, (NH * TL + L - 1) // L, tl5_body, 0)

    cps = [pltpu.async_copy(v, h.at[w], sem_out)
           for v, h in zip(out_v, out_hbm)]
    for c in cps:
        c.wait()


_sc_kernel = pl.kernel(
    _sc_body,
    out_type=tuple(jax.ShapeDtypeStruct((S, n), dt) for _, n, dt in _SC_OUT),
    mesh=plsc.VectorSubcoreMesh(core_axis_name="c", subcore_axis_name="s"),
    compiler_params=pltpu.CompilerParams(needs_layout_passes=False),
    scratch_types=(
        tuple(pltpu.VMEM((n,), dt) for _, n, dt in _SC_IN)
        + (pltpu.VMEM((TL, NH), _i32),)
        + tuple(pltpu.VMEM((n,), dt) for _, n, dt in _SC_OUT)
        + (pltpu.SemaphoreType.DMA, pltpu.SemaphoreType.DMA)
    ),
)


_PB = 256


def _map_body(mv_ref, mt_ref, mpos_ref, mdir_ref,
              opos_ref, odir_ref, otype_ref):
    mv = mv_ref[...].astype(_i32)
    for n in range(N):
        m = (mv[n] != 0)[:, None, :]
        opos_ref[:, n] = jnp.where(m, mpos_ref[:, n], 0.0)
        odir_ref[:, n] = jnp.where(m, mdir_ref[:, n], 0.0)
    anyv = jnp.max(mv, axis=0, keepdims=True) != 0
    otype_ref[...] = jnp.where(anyv, mt_ref[...].astype(_i32),
                               0).astype(jnp.int8)


_map_call = pl.pallas_call(
    _map_body,
    grid=(P // _PB,),
    in_specs=[
        pl.BlockSpec((N, S, _PB), lambda i: (0, 0, i)),
        pl.BlockSpec((11, S, _PB), lambda i: (0, 0, i)),
        pl.BlockSpec((S, N, 2, _PB), lambda i: (0, 0, 0, i)),
        pl.BlockSpec((S, N, 2, _PB), lambda i: (0, 0, 0, i)),
    ],
    out_specs=[
        pl.BlockSpec((S, N, 2, _PB), lambda i: (0, 0, 0, i)),
        pl.BlockSpec((S, N, 2, _PB), lambda i: (0, 0, 0, i)),
        pl.BlockSpec((11, S, _PB), lambda i: (0, 0, i)),
    ],
    out_shape=[
        jax.ShapeDtypeStruct((S, N, 2, P), _f32),
        jax.ShapeDtypeStruct((S, N, 2, P), _f32),
        jax.ShapeDtypeStruct((11, S, P), jnp.int8),
    ],
)


def kernel(agent_valid, agent_pos, agent_vel, agent_spd, agent_acc,
           agent_yaw_bbox, agent_yaw_rate, agent_type, agent_role,
           agent_size, agent_cmd, map_valid, map_type, map_pos, map_dir,
           tl_valid, tl_state, tl_pos, tl_dir):
    sc_in = (
        agent_valid.transpose(0, 2, 1).astype(_i32).reshape(S, A * T),
        agent_pos.transpose(0, 2, 3, 1).reshape(S, A * 2 * T),
        agent_vel.transpose(0, 2, 3, 1).reshape(S, A * 2 * T),
        agent_spd.transpose(0, 2, 3, 1).reshape(S, A * T),
        agent_acc.transpose(0, 2, 3, 1).reshape(S, A * T),
        agent_yaw_bbox.transpose(0, 2, 3, 1).reshape(S, A * T),
        agent_yaw_rate.transpose(0, 2, 3, 1).reshape(S, A * T),
        agent_type.astype(_i32).reshape(S, A * 3),
        agent_role.astype(_i32).reshape(S, A * 3),
        agent_size.reshape(S, A * 3),
        tl_valid.transpose(0, 2, 1).astype(_i32).reshape(S, TL * T),
        tl_pos.transpose(0, 2, 3, 1).reshape(S, TL * 2 * T),
        tl_dir.transpose(0, 2, 3, 1).reshape(S, TL * 2 * T),
        tl_state[:, :NH].transpose(0, 3, 2, 1).astype(_i32),
    )
    (o_refpos, o_refyaw, o_refrot, o_scval, o_scpos, o_scvel, o_scspd,
     o_scacc, o_scyaw, o_scyrt, o_gtval, o_gtpos, o_gtvel, o_gtspd,
     o_gtyaw, o_typ, o_rol, o_siz, o_tlp, o_tld, o_tls) = _sc_kernel(*sc_in)

    o_mpos, o_mdir, o_mtype = _map_call(
        map_valid.transpose(2, 0, 1).astype(jnp.int8),
        map_type.transpose(2, 0, 1).astype(jnp.int8),
        map_pos.transpose(0, 2, 3, 1),
        map_dir.transpose(0, 2, 3, 1),
    )

    return (
        o_refpos.reshape(S, A, 1, 2),
        o_refyaw.reshape(S, A, 1),
        o_refrot.reshape(S, A, 2, 2),
        agent_type,
        agent_role,
        o_gtval.astype(bool).reshape(S, A, NF),
        o_gtpos.reshape(S, A, NF, 2),
        o_gtspd.reshape(S, A, NF, 1),
        o_gtvel.reshape(S, A, NF, 2),
        o_gtyaw.reshape(S, A, NF, 1),
        agent_cmd,
        o_scval.astype(bool).reshape(S, A, NH),
        o_scpos.reshape(S, A, NH, 2),
        o_scvel.reshape(S, A, NH, 2),
        o_scspd.reshape(S, A, NH, 1),
        o_scacc.reshape(S, A, NH, 1),
        o_scyaw.reshape(S, A, NH, 1),
        o_scyrt.reshape(S, A, NH, 1),
        o_typ.astype(bool).reshape(S, A, 3),
        o_rol.astype(bool).reshape(S, A, 3),
        o_siz.reshape(S, A, 3),
        map_valid,
        o_mtype.transpose(1, 2, 0).astype(bool),
        o_mpos.transpose(0, 3, 1, 2),
        o_mdir.transpose(0, 3, 1, 2),
        tl_valid[:, :NH],
        o_tls[:, :NH * TL * 5].astype(bool).reshape(S, NH, TL, 5),
        o_tlp.reshape(S, NH, TL, 2),
        o_tld.reshape(S, NH, TL, 2),
    )


# speedup vs baseline: 1.9094x; 1.0673x over previous
import jax
import jax.numpy as jnp
from jax import lax
from jax.experimental import pallas as pl
from jax.experimental.pallas import tpu as pltpu
from jax.experimental.pallas import tpu_sc as plsc

S, T, A = 32, 91, 64
P, N = 1024, 20
TL = 40
NH = 11
NF = T - NH
NC, NS = 2, 16
L = 16

_f32 = jnp.float32
_i32 = jnp.int32

_SC_IN = (
    ("av", A * T, _i32),
    ("pos", A * 2 * T, _f32),
    ("vel", A * 2 * T, _f32),
    ("spd", A * T, _f32),
    ("acc", A * T, _f32),
    ("yaw", A * T, _f32),
    ("yrt", A * T, _f32),
    ("typ", A * 3, _i32),
    ("rol", A * 3, _i32),
    ("siz", A * 3, _f32),
    ("tlv", TL * T, _i32),
    ("tlp", TL * 2 * T, _f32),
    ("tld", TL * 2 * T, _f32),
)

_SC_OUT = (
    ("refpos", A * 2, _f32),
    ("refyaw", A, _f32),
    ("refrot", A * 4, _f32),
    ("scval", A * NH, _i32),
    ("scpos", A * NH * 2, _f32),
    ("scvel", A * NH * 2, _f32),
    ("scspd", A * NH, _f32),
    ("scacc", A * NH, _f32),
    ("scyaw", A * NH, _f32),
    ("scyrt", A * NH, _f32),
    ("gtval", A * NF, _i32),
    ("gtpos", A * NF * 2, _f32),
    ("gtvel", A * NF * 2, _f32),
    ("gtspd", A * NF, _f32),
    ("gtyaw", A * NF, _f32),
    ("otyp", A * 3, _i32),
    ("orol", A * 3, _i32),
    ("osiz", A * 3, _f32),
    ("otlp", NH * TL * 2, _f32),
    ("otld", NH * TL * 2, _f32),
    ("otls", NH * TL * 5 + 8, _i32),
)

_PI = 3.14159265358979
_TWO_PI = 2.0 * _PI
_SIN_C = (1.0, -1.0 / 6, 1.0 / 120, -1.0 / 5040, 1.0 / 362880,
          -1.0 / 39916800, 1.0 / 6227020800)
_COS_C = (1.0, -0.5, 1.0 / 24, -1.0 / 720, 1.0 / 40320, -1.0 / 3628800,
          1.0 / 479001600, -1.0 / 87178291200)


def _horner(x2, coeffs):
    acc = jnp.full(x2.shape, coeffs[-1], _f32)
    for k in reversed(coeffs[:-1]):
        acc = acc * x2 + _f32(k)
    return acc


def _wrap_pi(x):
    t = lax.rem(x + _f32(_PI), _f32(_TWO_PI))
    t = jnp.where((t != 0.0) & (t < 0.0), t + _f32(_TWO_PI), t)
    return t - _f32(_PI)


def _sincos(x):
    r = _wrap_pi(x)
    r2 = r * r
    return r * _horner(r2, _SIN_C), _horner(r2, _COS_C)


def _sc_body(*refs):
    n_in, n_out = len(_SC_IN) + 1, len(_SC_OUT)
    in_hbm = refs[:n_in]
    out_hbm = refs[n_in:n_in + n_out]
    in_v = refs[n_in + n_out:n_in + n_out + n_in]
    out_v = refs[n_in + n_out + n_in:n_in + n_out + n_in + n_out]
    sem_in, sem_out = refs[-2], refs[-1]

    w = lax.axis_index("s") * NC + lax.axis_index("c")

    tls_h = in_hbm[-1]
    cps = [pltpu.async_copy(h.at[w], v, sem_in)
           for h, v in zip(in_hbm[:-1], in_v[:-1])]
    for c in cps:
        c.wait()

    (av_v, pos_v, vel_v, spd_v, acc_v, yaw_v, yrt_v, typ_v, rol_v, siz_v,
     tlv_v, tlp_v, tld_v, tls_v) = in_v
    (orefpos, orefyaw, orefrot, oscval, oscpos, oscvel, oscspd, oscacc,
     oscyaw, oscyrt, ogtval, ogtpos, ogtvel, ogtspd, ogtyaw, otyp, orol,
     osiz, otlp, otld, otls) = out_v

    iota = lax.iota(_i32, L)

    for g in range(A // L):
        a16 = g * L + iota
        aT = a16 * T
        a2T = a16 * (2 * T)

        lvs = jnp.full((L,), NH - 1, _i32)
        anyv = jnp.zeros((L,), _i32)
        for t in range(NH):
            v = plsc.load_gather(av_v, [aT + t])
            lvs = jnp.where(v != 0, jnp.full((L,), t, _i32), lvs)
            anyv = anyv | v

        refyaw = plsc.load_gather(yaw_v, [aT + lvs])
        refpx = plsc.load_gather(pos_v, [a2T + lvs])
        refpy = plsc.load_gather(pos_v, [a2T + T + lvs])
        sn, cs = _sincos(refyaw)

        orefyaw[pl.ds(g * L, L)] = refyaw
        plsc.store_scatter(orefpos, [a16 * 2], refpx)
        plsc.store_scatter(orefpos, [a16 * 2 + 1], refpy)
        plsc.store_scatter(orefrot, [a16 * 4], cs)
        plsc.store_scatter(orefrot, [a16 * 4 + 1], -sn)
        plsc.store_scatter(orefrot, [a16 * 4 + 2], sn)
        plsc.store_scatter(orefrot, [a16 * 4 + 3], cs)

        for t in range(NH):
            vb = plsc.load_gather(av_v, [aT + t])
            vf = vb != 0
            plsc.store_scatter(oscval, [a16 * NH + t], vb)
            px = plsc.load_gather(pos_v, [a2T + t])
            py = plsc.load_gather(pos_v, [a2T + T + t])
            plsc.store_scatter(oscpos, [a16 * (2 * NH) + 2 * t],
                               jnp.where(vf, px, 0.0))
            plsc.store_scatter(oscpos, [a16 * (2 * NH) + 2 * t + 1],
                               jnp.where(vf, py, 0.0))
            vx = plsc.load_gather(vel_v, [a2T + t])
            vy = plsc.load_gather(vel_v, [a2T + T + t])
            plsc.store_scatter(oscvel, [a16 * (2 * NH) + 2 * t],
                               jnp.where(vf, vx, 0.0))
            plsc.store_scatter(oscvel, [a16 * (2 * NH) + 2 * t + 1],
                               jnp.where(vf, vy, 0.0))
            for src_v, dst_v in ((spd_v, oscspd), (acc_v, oscacc),
                                 (yaw_v, oscyaw), (yrt_v, oscyrt)):
                x = plsc.load_gather(src_v, [aT + t])
                plsc.store_scatter(dst_v, [a16 * NH + t],
                                   jnp.where(vf, x, 0.0))

        def gt_body(t, _, a16=a16, aT=aT, a2T=a2T, refpx=refpx, refpy=refpy,
                    refyaw=refyaw, sn=sn, cs=cs):
            tf = t - NH
            vb = plsc.load_gather(av_v, [aT + t])
            vf = vb != 0
            plsc.store_scatter(ogtval, [a16 * NF + tf], vb)
            px = plsc.load_gather(pos_v, [a2T + t])
            py = plsc.load_gather(pos_v, [a2T + T + t])
            rx = px - refpx
            ry = py - refpy
            plsc.store_scatter(ogtpos, [a16 * (2 * NF) + tf],
                               jnp.where(vf, rx * cs + ry * sn, 0.0))
            plsc.store_scatter(ogtpos, [a16 * (2 * NF) + NF + tf],
                               jnp.where(vf, ry * cs - rx * sn, 0.0))
            vx = plsc.load_gather(vel_v, [a2T + t])
            vy = plsc.load_gather(vel_v, [a2T + T + t])
            plsc.store_scatter(ogtvel, [a16 * (2 * NF) + tf],
                               jnp.where(vf, vx * cs + vy * sn, 0.0))
            plsc.store_scatter(ogtvel, [a16 * (2 * NF) + NF + tf],
                               jnp.where(vf, vy * cs - vx * sn, 0.0))
            sp = plsc.load_gather(spd_v, [aT + t])
            plsc.store_scatter(ogtspd, [a16 * NF + tf],
                               jnp.where(vf, sp, 0.0))
            yw = plsc.load_gather(yaw_v, [aT + t])
            plsc.store_scatter(ogtyaw, [a16 * NF + tf],
                               jnp.where(vf, _wrap_pi(yw - refyaw), 0.0))
            return 0

        lax.fori_loop(NH, T, gt_body, 0)

        af = anyv != 0
        for ch in range(3):
            tv_ = plsc.load_gather(typ_v, [a16 * 3 + ch])
            plsc.store_scatter(otyp, [a16 * 3 + ch], jnp.where(af, tv_, 0))
            rv = plsc.load_gather(rol_v, [a16 * 3 + ch])
            plsc.store_scatter(orol, [a16 * 3 + ch], jnp.where(af, rv, 0))
            sv = plsc.load_gather(siz_v, [a16 * 3 + ch])
            plsc.store_scatter(osiz, [a16 * 3 + ch], jnp.where(af, sv, 0.0))

    def tl2_body(i, _):
        j = i * L + iota
        t = j // (2 * TL)
        r = j - t * (2 * TL)
        l = r // 2
        c = r - l * 2
        m = plsc.load_gather(tlv_v, [l * T + t])
        mf = m != 0
        xp = plsc.load_gather(tlp_v, [l * (2 * T) + c * T + t])
        plsc.store_scatter(otlp, [j], jnp.where(mf, xp, 0.0))
        xd = plsc.load_gather(tld_v, [l * (2 * T) + c * T + t])
        plsc.store_scatter(otld, [j], jnp.where(mf, xd, 0.0))
        return 0

    lax.fori_loop(0, (NH * TL * 2) // L, tl2_body, 0)

    for q in range(5):
        pltpu.sync_copy(tls_h.at[w, q], tls_v)

        def tl5_body(i, _, q=q):
            k = i * L + iota
            t = k // TL
            l = k - t * TL
            m = plsc.load_gather(tlv_v, [l * T + t])
            x = plsc.load_gather(tls_v, [l, jnp.minimum(t, NH - 1)])
            jj = t * (TL * 5) + l * 5 + q
            plsc.store_scatter(otls, [jj], jnp.where(m != 0, x, 0),
                               mask=k < NH * TL)
            return 0

        lax.fori_loop(0, (NH * TL + L - 1) // L, tl5_body, 0)

    cps = [pltpu.async_copy(v, h.at[w], sem_out)
           for v, h in zip(out_v, out_hbm)]
    for c in cps:
        c.wait()


_sc_kernel = pl.kernel(
    _sc_body,
    out_type=tuple(jax.ShapeDtypeStruct((S, n), dt) for _, n, dt in _SC_OUT),
    mesh=plsc.VectorSubcoreMesh(core_axis_name="c", subcore_axis_name="s"),
    compiler_params=pltpu.CompilerParams(needs_layout_passes=False),
    scratch_types=(
        tuple(pltpu.VMEM((n,), dt) for _, n, dt in _SC_IN)
        + (pltpu.VMEM((TL, NH), _i32),)
        + tuple(pltpu.VMEM((n,), dt) for _, n, dt in _SC_OUT)
        + (pltpu.SemaphoreType.DMA, pltpu.SemaphoreType.DMA)
    ),
)


_PB = 256


def _map_body(mv_ref, mt_ref, mpos_ref, mdir_ref,
              opos_ref, odir_ref, otype_ref):
    mv = mv_ref[...]
    for n in range(N):
        m = mv[n][:, None, :]
        opos_ref[:, n] = jnp.where(m, mpos_ref[:, n], 0.0)
        odir_ref[:, n] = jnp.where(m, mdir_ref[:, n], 0.0)
    anyv = jnp.any(mv, axis=0, keepdims=True)
    otype_ref[...] = jnp.logical_and(anyv, mt_ref[...])


_map_call = pl.pallas_call(
    _map_body,
    grid=(P // _PB,),
    in_specs=[
        pl.BlockSpec((N, S, _PB), lambda i: (0, 0, i)),
        pl.BlockSpec((11, S, _PB), lambda i: (0, 0, i)),
        pl.BlockSpec((S, N, 2, _PB), lambda i: (0, 0, 0, i)),
        pl.BlockSpec((S, N, 2, _PB), lambda i: (0, 0, 0, i)),
    ],
    out_specs=[
        pl.BlockSpec((S, N, 2, _PB), lambda i: (0, 0, 0, i)),
        pl.BlockSpec((S, N, 2, _PB), lambda i: (0, 0, 0, i)),
        pl.BlockSpec((11, S, _PB), lambda i: (0, 0, i)),
    ],
    out_shape=[
        jax.ShapeDtypeStruct((S, N, 2, P), _f32),
        jax.ShapeDtypeStruct((S, N, 2, P), _f32),
        jax.ShapeDtypeStruct((11, S, P), jnp.bool_),
    ],
)


def kernel(agent_valid, agent_pos, agent_vel, agent_spd, agent_acc,
           agent_yaw_bbox, agent_yaw_rate, agent_type, agent_role,
           agent_size, agent_cmd, map_valid, map_type, map_pos, map_dir,
           tl_valid, tl_state, tl_pos, tl_dir):
    sc_in = (
        agent_valid.transpose(0, 2, 1).astype(_i32).reshape(S, A * T),
        agent_pos.transpose(0, 2, 3, 1).reshape(S, A * 2 * T),
        agent_vel.transpose(0, 2, 3, 1).reshape(S, A * 2 * T),
        agent_spd.transpose(0, 2, 3, 1).reshape(S, A * T),
        agent_acc.transpose(0, 2, 3, 1).reshape(S, A * T),
        agent_yaw_bbox.transpose(0, 2, 3, 1).reshape(S, A * T),
        agent_yaw_rate.transpose(0, 2, 3, 1).reshape(S, A * T),
        agent_type.astype(_i32).reshape(S, A * 3),
        agent_role.astype(_i32).reshape(S, A * 3),
        agent_size.reshape(S, A * 3),
        tl_valid.transpose(0, 2, 1).astype(_i32).reshape(S, TL * T),
        tl_pos.transpose(0, 2, 3, 1).reshape(S, TL * 2 * T),
        tl_dir.transpose(0, 2, 3, 1).reshape(S, TL * 2 * T),
        tl_state[:, :NH].transpose(0, 3, 2, 1).astype(_i32),
    )
    (o_refpos, o_refyaw, o_refrot, o_scval, o_scpos, o_scvel, o_scspd,
     o_scacc, o_scyaw, o_scyrt, o_gtval, o_gtpos, o_gtvel, o_gtspd,
     o_gtyaw, o_typ, o_rol, o_siz, o_tlp, o_tld, o_tls) = _sc_kernel(*sc_in)

    o_mpos, o_mdir, o_mtype = _map_call(
        map_valid.transpose(2, 0, 1),
        map_type.transpose(2, 0, 1),
        map_pos.transpose(0, 2, 3, 1),
        map_dir.transpose(0, 2, 3, 1),
    )

    return (
        o_refpos.reshape(S, A, 1, 2),
        o_refyaw.reshape(S, A, 1),
        o_refrot.reshape(S, A, 2, 2),
        agent_type,
        agent_role,
        o_gtval.astype(bool).reshape(S, A, NF),
        o_gtpos.reshape(S, A, 2, NF).transpose(0, 1, 3, 2),
        o_gtspd.reshape(S, A, NF, 1),
        o_gtvel.reshape(S, A, 2, NF).transpose(0, 1, 3, 2),
        o_gtyaw.reshape(S, A, NF, 1),
        agent_cmd,
        o_scval.astype(bool).reshape(S, A, NH),
        o_scpos.reshape(S, A, NH, 2),
        o_scvel.reshape(S, A, NH, 2),
        o_scspd.reshape(S, A, NH, 1),
        o_scacc.reshape(S, A, NH, 1),
        o_scyaw.reshape(S, A, NH, 1),
        o_scyrt.reshape(S, A, NH, 1),
        o_typ.astype(bool).reshape(S, A, 3),
        o_rol.astype(bool).reshape(S, A, 3),
        o_siz.reshape(S, A, 3),
        map_valid,
        o_mtype.transpose(1, 2, 0),
        o_mpos.transpose(0, 3, 1, 2),
        o_mdir.transpose(0, 3, 1, 2),
        tl_valid[:, :NH],
        o_tls[:, :NH * TL * 5].astype(bool).reshape(S, NH, TL, 5),
        o_tlp.reshape(S, NH, TL, 2),
        o_tld.reshape(S, NH, TL, 2),
    )


# speedup vs baseline: 2.1156x; 1.1080x over previous
import jax
import jax.numpy as jnp
from jax import lax
from jax.experimental import pallas as pl
from jax.experimental.pallas import tpu as pltpu
from jax.experimental.pallas import tpu_sc as plsc

S, T, A = 32, 91, 64
P, N = 1024, 20
TL = 40
NH = 11
NF = T - NH
NC, NS = 2, 16
L = 16

_f32 = jnp.float32
_i32 = jnp.int32

_SC_IN = (
    ("av", A * T, _i32),
    ("pos", A * 2 * T, _f32),
    ("vel", A * 2 * T, _f32),
    ("spd", A * T, _f32),
    ("acc", A * T, _f32),
    ("yaw", A * T, _f32),
    ("yrt", A * T, _f32),
    ("typ", A * 3, _i32),
    ("rol", A * 3, _i32),
    ("siz", A * 3, _f32),
    ("tlv", TL * NH, _i32),
    ("tlp", TL * 2 * NH, _f32),
    ("tld", TL * 2 * NH, _f32),
)

_SC_OUT = (
    ("refpos", (1, 2, A), _f32),
    ("refyaw", (1, A), _f32),
    ("refrot", (2, 2, A), _f32),
    ("scval", (A * NH,), _i32),
    ("scpos", (NH, 2, A), _f32),
    ("scvel", (NH, 2, A), _f32),
    ("scspd", (NH, 1, A), _f32),
    ("scacc", (NH, 1, A), _f32),
    ("scyaw", (NH, 1, A), _f32),
    ("scyrt", (NH, 1, A), _f32),
    ("gtval", (A * NF,), _i32),
    ("gtpos", (A * NF * 2,), _f32),
    ("gtvel", (A * NF * 2,), _f32),
    ("gtspd", (A * NF,), _f32),
    ("gtyaw", (A * NF,), _f32),
    ("otyp", (A * 3,), _i32),
    ("orol", (A * 3,), _i32),
    ("osiz", (A * 3,), _f32),
    ("otlp", (NH, 2, TL), _f32),
    ("otld", (NH, 2, TL), _f32),
    ("otls", (NH * TL * 5 + 8,), _i32),
)

_PI = 3.14159265358979
_TWO_PI = 2.0 * _PI
_SIN_C = (1.0, -1.0 / 6, 1.0 / 120, -1.0 / 5040, 1.0 / 362880,
          -1.0 / 39916800, 1.0 / 6227020800)
_COS_C = (1.0, -0.5, 1.0 / 24, -1.0 / 720, 1.0 / 40320, -1.0 / 3628800,
          1.0 / 479001600, -1.0 / 87178291200)


def _horner(x2, coeffs):
    acc = jnp.full(x2.shape, coeffs[-1], _f32)
    for k in reversed(coeffs[:-1]):
        acc = acc * x2 + _f32(k)
    return acc


def _wrap_pi(x):
    t = lax.rem(x + _f32(_PI), _f32(_TWO_PI))
    t = jnp.where((t != 0.0) & (t < 0.0), t + _f32(_TWO_PI), t)
    return t - _f32(_PI)


def _sincos(x):
    r = _wrap_pi(x)
    r2 = r * r
    return r * _horner(r2, _SIN_C), _horner(r2, _COS_C)


def _sc_body(*refs):
    n_in, n_out = len(_SC_IN) + 1, len(_SC_OUT)
    in_hbm = refs[:n_in]
    out_hbm = refs[n_in:n_in + n_out]
    in_v = refs[n_in + n_out:n_in + n_out + n_in]
    out_v = refs[n_in + n_out + n_in:n_in + n_out + n_in + n_out]
    sem_in, sem_out = refs[-2], refs[-1]

    w = lax.axis_index("s") * NC + lax.axis_index("c")

    tls_h = in_hbm[-1]
    cps = [pltpu.async_copy(h.at[w], v, sem_in)
           for h, v in zip(in_hbm[:-1], in_v[:-1])]
    for c in cps:
        c.wait()

    (av_v, pos_v, vel_v, spd_v, acc_v, yaw_v, yrt_v, typ_v, rol_v, siz_v,
     tlv_v, tlp_v, tld_v, tls_v) = in_v
    (orefpos, orefyaw, orefrot, oscval, oscpos, oscvel, oscspd, oscacc,
     oscyaw, oscyrt, ogtval, ogtpos, ogtvel, ogtspd, ogtyaw, otyp, orol,
     osiz, otlp, otld, otls) = out_v

    iota = lax.iota(_i32, L)

    for g in range(A // L):
        a16 = g * L + iota
        aT = a16 * T
        a2T = a16 * (2 * T)

        lvs = jnp.full((L,), NH - 1, _i32)
        anyv = jnp.zeros((L,), _i32)
        for t in range(NH):
            v = plsc.load_gather(av_v, [aT + t])
            lvs = jnp.where(v != 0, jnp.full((L,), t, _i32), lvs)
            anyv = anyv | v

        refyaw = plsc.load_gather(yaw_v, [aT + lvs])
        refpx = plsc.load_gather(pos_v, [a2T + lvs])
        refpy = plsc.load_gather(pos_v, [a2T + T + lvs])
        sn, cs = _sincos(refyaw)

        orefyaw[0, pl.ds(g * L, L)] = refyaw
        orefpos[0, 0, pl.ds(g * L, L)] = refpx
        orefpos[0, 1, pl.ds(g * L, L)] = refpy
        orefrot[0, 0, pl.ds(g * L, L)] = cs
        orefrot[0, 1, pl.ds(g * L, L)] = -sn
        orefrot[1, 0, pl.ds(g * L, L)] = sn
        orefrot[1, 1, pl.ds(g * L, L)] = cs

        for t in range(NH):
            vb = plsc.load_gather(av_v, [aT + t])
            vf = vb != 0
            plsc.store_scatter(oscval, [a16 * NH + t], vb)
            px = plsc.load_gather(pos_v, [a2T + t])
            py = plsc.load_gather(pos_v, [a2T + T + t])
            oscpos[t, 0, pl.ds(g * L, L)] = jnp.where(vf, px, 0.0)
            oscpos[t, 1, pl.ds(g * L, L)] = jnp.where(vf, py, 0.0)
            vx = plsc.load_gather(vel_v, [a2T + t])
            vy = plsc.load_gather(vel_v, [a2T + T + t])
            oscvel[t, 0, pl.ds(g * L, L)] = jnp.where(vf, vx, 0.0)
            oscvel[t, 1, pl.ds(g * L, L)] = jnp.where(vf, vy, 0.0)
            for src_v, dst_v in ((spd_v, oscspd), (acc_v, oscacc),
                                 (yaw_v, oscyaw), (yrt_v, oscyrt)):
                x = plsc.load_gather(src_v, [aT + t])
                dst_v[t, 0, pl.ds(g * L, L)] = jnp.where(vf, x, 0.0)

        def gt_body(t, _, a16=a16, aT=aT, a2T=a2T, refpx=refpx, refpy=refpy,
                    refyaw=refyaw, sn=sn, cs=cs):
            tf = t - NH
            vb = plsc.load_gather(av_v, [aT + t])
            vf = vb != 0
            plsc.store_scatter(ogtval, [a16 * NF + tf], vb)
            px = plsc.load_gather(pos_v, [a2T + t])
            py = plsc.load_gather(pos_v, [a2T + T + t])
            rx = px - refpx
            ry = py - refpy
            plsc.store_scatter(ogtpos, [a16 * (2 * NF) + tf],
                               jnp.where(vf, rx * cs + ry * sn, 0.0))
            plsc.store_scatter(ogtpos, [a16 * (2 * NF) + NF + tf],
                               jnp.where(vf, ry * cs - rx * sn, 0.0))
            vx = plsc.load_gather(vel_v, [a2T + t])
            vy = plsc.load_gather(vel_v, [a2T + T + t])
            plsc.store_scatter(ogtvel, [a16 * (2 * NF) + tf],
                               jnp.where(vf, vx * cs + vy * sn, 0.0))
            plsc.store_scatter(ogtvel, [a16 * (2 * NF) + NF + tf],
                               jnp.where(vf, vy * cs - vx * sn, 0.0))
            sp = plsc.load_gather(spd_v, [aT + t])
            plsc.store_scatter(ogtspd, [a16 * NF + tf],
                               jnp.where(vf, sp, 0.0))
            yw = plsc.load_gather(yaw_v, [aT + t])
            plsc.store_scatter(ogtyaw, [a16 * NF + tf],
                               jnp.where(vf, _wrap_pi(yw - refyaw), 0.0))
            return 0

        lax.fori_loop(NH, T, gt_body, 0)

        af = anyv != 0
        for ch in range(3):
            tv_ = plsc.load_gather(typ_v, [a16 * 3 + ch])
            plsc.store_scatter(otyp, [a16 * 3 + ch], jnp.where(af, tv_, 0))
            rv = plsc.load_gather(rol_v, [a16 * 3 + ch])
            plsc.store_scatter(orol, [a16 * 3 + ch], jnp.where(af, rv, 0))
            sv = plsc.load_gather(siz_v, [a16 * 3 + ch])
            plsc.store_scatter(osiz, [a16 * 3 + ch], jnp.where(af, sv, 0.0))

    def tl2_body(i, _):
        j = i * L + iota
        t = j // (2 * TL)
        r = j - t * (2 * TL)
        l = r // 2
        c = r - l * 2
        m = plsc.load_gather(tlv_v, [l * NH + t])
        mf = m != 0
        xp = plsc.load_gather(tlp_v, [c * (TL * NH) + l * NH + t])
        plsc.store_scatter(otlp, [t, c, l], jnp.where(mf, xp, 0.0))
        xd = plsc.load_gather(tld_v, [c * (TL * NH) + l * NH + t])
        plsc.store_scatter(otld, [t, c, l], jnp.where(mf, xd, 0.0))
        return 0

    lax.fori_loop(0, (NH * TL * 2) // L, tl2_body, 0)

    for q in range(5):
        pltpu.sync_copy(tls_h.at[w, q], tls_v)

        def tl5_body(i, _, q=q):
            k = i * L + iota
            t = k // TL
            l = k - t * TL
            m = plsc.load_gather(tlv_v, [l * NH + jnp.minimum(t, NH - 1)])
            x = plsc.load_gather(tls_v, [l, jnp.minimum(t, NH - 1)])
            jj = t * (TL * 5) + l * 5 + q
            plsc.store_scatter(otls, [jj], jnp.where(m != 0, x, 0),
                               mask=k < NH * TL)
            return 0

        lax.fori_loop(0, (NH * TL + L - 1) // L, tl5_body, 0)

    cps = [pltpu.async_copy(v, h.at[w], sem_out)
           for v, h in zip(out_v, out_hbm)]
    for c in cps:
        c.wait()


_sc_kernel = pl.kernel(
    _sc_body,
    out_type=tuple(jax.ShapeDtypeStruct((S,) + sh, dt)
                   for _, sh, dt in _SC_OUT),
    mesh=plsc.VectorSubcoreMesh(core_axis_name="c", subcore_axis_name="s"),
    compiler_params=pltpu.CompilerParams(needs_layout_passes=False),
    scratch_types=(
        tuple(pltpu.VMEM((n,), dt) for _, n, dt in _SC_IN)
        + (pltpu.VMEM((TL, NH), _i32),)
        + tuple(pltpu.VMEM(sh, dt) for _, sh, dt in _SC_OUT)
        + (pltpu.SemaphoreType.DMA, pltpu.SemaphoreType.DMA)
    ),
)


_PB = 256


def _map_body(mv_ref, mt_ref, mpos_ref, mdir_ref,
              opos_ref, odir_ref, otype_ref):
    mv = mv_ref[...]
    for n in range(N):
        m = mv[n][:, None, :]
        opos_ref[:, n] = jnp.where(m, mpos_ref[:, n], 0.0)
        odir_ref[:, n] = jnp.where(m, mdir_ref[:, n], 0.0)
    anyv = jnp.any(mv, axis=0, keepdims=True)
    otype_ref[...] = jnp.logical_and(anyv, mt_ref[...])


_map_call = pl.pallas_call(
    _map_body,
    grid=(P // _PB,),
    in_specs=[
        pl.BlockSpec((N, S, _PB), lambda i: (0, 0, i)),
        pl.BlockSpec((11, S, _PB), lambda i: (0, 0, i)),
        pl.BlockSpec((S, N, 2, _PB), lambda i: (0, 0, 0, i)),
        pl.BlockSpec((S, N, 2, _PB), lambda i: (0, 0, 0, i)),
    ],
    out_specs=[
        pl.BlockSpec((S, N, 2, _PB), lambda i: (0, 0, 0, i)),
        pl.BlockSpec((S, N, 2, _PB), lambda i: (0, 0, 0, i)),
        pl.BlockSpec((11, S, _PB), lambda i: (0, 0, i)),
    ],
    out_shape=[
        jax.ShapeDtypeStruct((S, N, 2, P), _f32),
        jax.ShapeDtypeStruct((S, N, 2, P), _f32),
        jax.ShapeDtypeStruct((11, S, P), jnp.bool_),
    ],
)


def kernel(agent_valid, agent_pos, agent_vel, agent_spd, agent_acc,
           agent_yaw_bbox, agent_yaw_rate, agent_type, agent_role,
           agent_size, agent_cmd, map_valid, map_type, map_pos, map_dir,
           tl_valid, tl_state, tl_pos, tl_dir):
    sc_in = (
        agent_valid.transpose(0, 2, 1).astype(_i32).reshape(S, A * T),
        agent_pos.transpose(0, 2, 3, 1).reshape(S, A * 2 * T),
        agent_vel.transpose(0, 2, 3, 1).reshape(S, A * 2 * T),
        agent_spd.transpose(0, 2, 3, 1).reshape(S, A * T),
        agent_acc.transpose(0, 2, 3, 1).reshape(S, A * T),
        agent_yaw_bbox.transpose(0, 2, 3, 1).reshape(S, A * T),
        agent_yaw_rate.transpose(0, 2, 3, 1).reshape(S, A * T),
        agent_type.astype(_i32).reshape(S, A * 3),
        agent_role.astype(_i32).reshape(S, A * 3),
        agent_size.reshape(S, A * 3),
        tl_valid[:, :NH].transpose(0, 2, 1).astype(_i32).reshape(S, TL * NH),
        tl_pos[:, :NH].transpose(0, 3, 2, 1).reshape(S, TL * 2 * NH),
        tl_dir[:, :NH].transpose(0, 3, 2, 1).reshape(S, TL * 2 * NH),
        tl_state[:, :NH].transpose(0, 3, 2, 1).astype(_i32),
    )
    (o_refpos, o_refyaw, o_refrot, o_scval, o_scpos, o_scvel, o_scspd,
     o_scacc, o_scyaw, o_scyrt, o_gtval, o_gtpos, o_gtvel, o_gtspd,
     o_gtyaw, o_typ, o_rol, o_siz, o_tlp, o_tld, o_tls) = _sc_kernel(*sc_in)

    o_mpos, o_mdir, o_mtype = _map_call(
        map_valid.transpose(2, 0, 1),
        map_type.transpose(2, 0, 1),
        map_pos.transpose(0, 2, 3, 1),
        map_dir.transpose(0, 2, 3, 1),
    )

    return (
        o_refpos.transpose(0, 3, 1, 2),
        o_refyaw.transpose(0, 2, 1),
        o_refrot.transpose(0, 3, 1, 2),
        agent_type,
        agent_role,
        o_gtval.astype(bool).reshape(S, A, NF),
        o_gtpos.reshape(S, A, 2, NF).transpose(0, 1, 3, 2),
        o_gtspd.reshape(S, A, NF, 1),
        o_gtvel.reshape(S, A, 2, NF).transpose(0, 1, 3, 2),
        o_gtyaw.reshape(S, A, NF, 1),
        agent_cmd,
        o_scval.astype(bool).reshape(S, A, NH),
        o_scpos.transpose(0, 3, 1, 2),
        o_scvel.transpose(0, 3, 1, 2),
        o_scspd.transpose(0, 3, 1, 2),
        o_scacc.transpose(0, 3, 1, 2),
        o_scyaw.transpose(0, 3, 1, 2),
        o_scyrt.transpose(0, 3, 1, 2),
        o_typ.astype(bool).reshape(S, A, 3),
        o_rol.astype(bool).reshape(S, A, 3),
        o_siz.reshape(S, A, 3),
        map_valid,
        o_mtype.transpose(1, 2, 0),
        o_mpos.transpose(0, 3, 1, 2),
        o_mdir.transpose(0, 3, 1, 2),
        tl_valid[:, :NH],
        o_tls[:, :NH * TL * 5].astype(bool).reshape(S, NH, TL, 5),
        o_tlp.transpose(0, 1, 3, 2),
        o_tld.transpose(0, 1, 3, 2),
    )


# speedup vs baseline: 2.1605x; 1.0212x over previous
import jax
import jax.numpy as jnp
from jax import lax
from jax.experimental import pallas as pl
from jax.experimental.pallas import tpu as pltpu
from jax.experimental.pallas import tpu_sc as plsc

S, T, A = 32, 91, 64
P, N = 1024, 20
TL = 40
NH = 11
NF = T - NH
NC, NS = 2, 16
L = 16

_f32 = jnp.float32
_i32 = jnp.int32

_SC_IN = (
    ("av", A * T, _i32),
    ("pos", A * 2 * T, _f32),
    ("vel", A * 2 * T, _f32),
    ("spd", A * T, _f32),
    ("acc", A * T, _f32),
    ("yaw", A * T, _f32),
    ("yrt", A * T, _f32),
    ("typ", A * 3, _i32),
    ("rol", A * 3, _i32),
    ("siz", A * 3, _f32),
    ("tlv", TL * NH, _i32),
    ("tlp", TL * 2 * NH, _f32),
    ("tld", TL * 2 * NH, _f32),
    ("tls", TL * 5 * NH, _i32),
)

_SC_OUT = (
    ("refpos", (1, 2, A), _f32),
    ("refyaw", (1, A), _f32),
    ("refrot", (2, 2, A), _f32),
    ("scval", (A * NH,), _i32),
    ("scpos", (NH, 2, A), _f32),
    ("scvel", (NH, 2, A), _f32),
    ("scspd", (NH, 1, A), _f32),
    ("scacc", (NH, 1, A), _f32),
    ("scyaw", (NH, 1, A), _f32),
    ("scyrt", (NH, 1, A), _f32),
    ("gtval", (A * NF,), _i32),
    ("gtpos", (A * NF * 2,), _f32),
    ("gtvel", (A * NF * 2,), _f32),
    ("gtspd", (A * NF,), _f32),
    ("gtyaw", (A * NF,), _f32),
    ("otyp", (A * 3,), _i32),
    ("orol", (A * 3,), _i32),
    ("osiz", (A * 3,), _f32),
    ("otlp", (NH, 2, TL), _f32),
    ("otld", (NH, 2, TL), _f32),
    ("otls", (NH * TL * 5 + 8,), _i32),
)

_PI = 3.14159265358979
_TWO_PI = 2.0 * _PI
_SIN_C = (1.0, -1.0 / 6, 1.0 / 120, -1.0 / 5040, 1.0 / 362880,
          -1.0 / 39916800, 1.0 / 6227020800)
_COS_C = (1.0, -0.5, 1.0 / 24, -1.0 / 720, 1.0 / 40320, -1.0 / 3628800,
          1.0 / 479001600, -1.0 / 87178291200)


def _horner(x2, coeffs):
    acc = jnp.full(x2.shape, coeffs[-1], _f32)
    for k in reversed(coeffs[:-1]):
        acc = acc * x2 + _f32(k)
    return acc


def _wrap_pi(x):
    t = lax.rem(x + _f32(_PI), _f32(_TWO_PI))
    t = jnp.where((t != 0.0) & (t < 0.0), t + _f32(_TWO_PI), t)
    return t - _f32(_PI)


def _sincos(x):
    r = _wrap_pi(x)
    r2 = r * r
    return r * _horner(r2, _SIN_C), _horner(r2, _COS_C)


def _sc_body(*refs):
    n_in, n_out = len(_SC_IN), len(_SC_OUT)
    in_hbm = refs[:n_in]
    out_hbm = refs[n_in:n_in + n_out]
    in_v = refs[n_in + n_out:n_in + n_out + n_in]
    out_v = refs[n_in + n_out + n_in:n_in + n_out + n_in + n_out]
    sem_in, sem_out = refs[-2], refs[-1]

    w = lax.axis_index("s") * NC + lax.axis_index("c")

    cps = [pltpu.async_copy(h.at[w], v, sem_in)
           for h, v in zip(in_hbm, in_v)]
    for c in cps:
        c.wait()

    (av_v, pos_v, vel_v, spd_v, acc_v, yaw_v, yrt_v, typ_v, rol_v, siz_v,
     tlv_v, tlp_v, tld_v, tls_v) = in_v
    (orefpos, orefyaw, orefrot, oscval, oscpos, oscvel, oscspd, oscacc,
     oscyaw, oscyrt, ogtval, ogtpos, ogtvel, ogtspd, ogtyaw, otyp, orol,
     osiz, otlp, otld, otls) = out_v

    iota = lax.iota(_i32, L)

    for g in range(A // L):
        a16 = g * L + iota
        aT = a16 * T
        a2T = a16 * (2 * T)

        lvs = jnp.full((L,), NH - 1, _i32)
        anyv = jnp.zeros((L,), _i32)
        for t in range(NH):
            v = plsc.load_gather(av_v, [aT + t])
            lvs = jnp.where(v != 0, jnp.full((L,), t, _i32), lvs)
            anyv = anyv | v

        refyaw = plsc.load_gather(yaw_v, [aT + lvs])
        refpx = plsc.load_gather(pos_v, [a2T + lvs])
        refpy = plsc.load_gather(pos_v, [a2T + T + lvs])
        sn, cs = _sincos(refyaw)

        orefyaw[0, pl.ds(g * L, L)] = refyaw
        orefpos[0, 0, pl.ds(g * L, L)] = refpx
        orefpos[0, 1, pl.ds(g * L, L)] = refpy
        orefrot[0, 0, pl.ds(g * L, L)] = cs
        orefrot[0, 1, pl.ds(g * L, L)] = -sn
        orefrot[1, 0, pl.ds(g * L, L)] = sn
        orefrot[1, 1, pl.ds(g * L, L)] = cs

        for t in range(NH):
            vb = plsc.load_gather(av_v, [aT + t])
            vf = vb != 0
            plsc.store_scatter(oscval, [a16 * NH + t], vb)
            px = plsc.load_gather(pos_v, [a2T + t])
            py = plsc.load_gather(pos_v, [a2T + T + t])
            oscpos[t, 0, pl.ds(g * L, L)] = jnp.where(vf, px, 0.0)
            oscpos[t, 1, pl.ds(g * L, L)] = jnp.where(vf, py, 0.0)
            vx = plsc.load_gather(vel_v, [a2T + t])
            vy = plsc.load_gather(vel_v, [a2T + T + t])
            oscvel[t, 0, pl.ds(g * L, L)] = jnp.where(vf, vx, 0.0)
            oscvel[t, 1, pl.ds(g * L, L)] = jnp.where(vf, vy, 0.0)
            for src_v, dst_v in ((spd_v, oscspd), (acc_v, oscacc),
                                 (yaw_v, oscyaw), (yrt_v, oscyrt)):
                x = plsc.load_gather(src_v, [aT + t])
                dst_v[t, 0, pl.ds(g * L, L)] = jnp.where(vf, x, 0.0)

        def gt_body(t, _, a16=a16, aT=aT, a2T=a2T, refpx=refpx, refpy=refpy,
                    refyaw=refyaw, sn=sn, cs=cs):
            tf = t - NH
            vb = plsc.load_gather(av_v, [aT + t])
            vf = vb != 0
            plsc.store_scatter(ogtval, [a16 * NF + tf], vb)
            px = plsc.load_gather(pos_v, [a2T + t])
            py = plsc.load_gather(pos_v, [a2T + T + t])
            rx = px - refpx
            ry = py - refpy
            plsc.store_scatter(ogtpos, [a16 * (2 * NF) + tf],
                               jnp.where(vf, rx * cs + ry * sn, 0.0))
            plsc.store_scatter(ogtpos, [a16 * (2 * NF) + NF + tf],
                               jnp.where(vf, ry * cs - rx * sn, 0.0))
            vx = plsc.load_gather(vel_v, [a2T + t])
            vy = plsc.load_gather(vel_v, [a2T + T + t])
            plsc.store_scatter(ogtvel, [a16 * (2 * NF) + tf],
                               jnp.where(vf, vx * cs + vy * sn, 0.0))
            plsc.store_scatter(ogtvel, [a16 * (2 * NF) + NF + tf],
                               jnp.where(vf, vy * cs - vx * sn, 0.0))
            sp = plsc.load_gather(spd_v, [aT + t])
            plsc.store_scatter(ogtspd, [a16 * NF + tf],
                               jnp.where(vf, sp, 0.0))
            yw = plsc.load_gather(yaw_v, [aT + t])
            plsc.store_scatter(ogtyaw, [a16 * NF + tf],
                               jnp.where(vf, _wrap_pi(yw - refyaw), 0.0))
            return 0

        lax.fori_loop(NH, T, gt_body, 0)

        af = anyv != 0
        for ch in range(3):
            tv_ = plsc.load_gather(typ_v, [a16 * 3 + ch])
            plsc.store_scatter(otyp, [a16 * 3 + ch], jnp.where(af, tv_, 0))
            rv = plsc.load_gather(rol_v, [a16 * 3 + ch])
            plsc.store_scatter(orol, [a16 * 3 + ch], jnp.where(af, rv, 0))
            sv = plsc.load_gather(siz_v, [a16 * 3 + ch])
            plsc.store_scatter(osiz, [a16 * 3 + ch], jnp.where(af, sv, 0.0))

    def tl2_body(i, _):
        j = i * L + iota
        t = j // (2 * TL)
        r = j - t * (2 * TL)
        l = r // 2
        c = r - l * 2
        m = plsc.load_gather(tlv_v, [l * NH + t])
        mf = m != 0
        xp = plsc.load_gather(tlp_v, [c * (TL * NH) + l * NH + t])
        plsc.store_scatter(otlp, [t, c, l], jnp.where(mf, xp, 0.0))
        xd = plsc.load_gather(tld_v, [c * (TL * NH) + l * NH + t])
        plsc.store_scatter(otld, [t, c, l], jnp.where(mf, xd, 0.0))
        return 0

    lax.fori_loop(0, (NH * TL * 2) // L, tl2_body, 0)

    def tl5_body(i, _):
        j = i * L + iota
        t = j // (TL * 5)
        r = j - t * (TL * 5)
        l = r // 5
        q = r - l * 5
        tc = jnp.minimum(t, NH - 1)
        m = plsc.load_gather(tlv_v, [l * NH + tc])
        x = plsc.load_gather(tls_v, [q * (TL * NH) + l * NH + tc])
        plsc.store_scatter(otls, [j], jnp.where(m != 0, x, 0))
        return 0

    lax.fori_loop(0, (NH * TL * 5 + 8) // L, tl5_body, 0)

    cps = [pltpu.async_copy(v, h.at[w], sem_out)
           for v, h in zip(out_v, out_hbm)]
    for c in cps:
        c.wait()


_sc_kernel = pl.kernel(
    _sc_body,
    out_type=tuple(jax.ShapeDtypeStruct((S,) + sh, dt)
                   for _, sh, dt in _SC_OUT),
    mesh=plsc.VectorSubcoreMesh(core_axis_name="c", subcore_axis_name="s"),
    compiler_params=pltpu.CompilerParams(needs_layout_passes=False),
    scratch_types=(
        tuple(pltpu.VMEM((n,), dt) for _, n, dt in _SC_IN)
        + tuple(pltpu.VMEM(sh, dt) for _, sh, dt in _SC_OUT)
        + (pltpu.SemaphoreType.DMA, pltpu.SemaphoreType.DMA)
    ),
)


_PB = 256


def _map_body(mv_ref, mt_ref, mpos_ref, mdir_ref,
              opos_ref, odir_ref, otype_ref):
    mv = mv_ref[...]
    for n in range(N):
        m = mv[n][:, None, :]
        opos_ref[:, n] = jnp.where(m, mpos_ref[:, n], 0.0)
        odir_ref[:, n] = jnp.where(m, mdir_ref[:, n], 0.0)
    anyv = jnp.any(mv, axis=0, keepdims=True)
    otype_ref[...] = jnp.logical_and(anyv, mt_ref[...])


_map_call = pl.pallas_call(
    _map_body,
    grid=(P // _PB,),
    in_specs=[
        pl.BlockSpec((N, S, _PB), lambda i: (0, 0, i)),
        pl.BlockSpec((11, S, _PB), lambda i: (0, 0, i)),
        pl.BlockSpec((S, N, 2, _PB), lambda i: (0, 0, 0, i)),
        pl.BlockSpec((S, N, 2, _PB), lambda i: (0, 0, 0, i)),
    ],
    out_specs=[
        pl.BlockSpec((S, N, 2, _PB), lambda i: (0, 0, 0, i)),
        pl.BlockSpec((S, N, 2, _PB), lambda i: (0, 0, 0, i)),
        pl.BlockSpec((11, S, _PB), lambda i: (0, 0, i)),
    ],
    out_shape=[
        jax.ShapeDtypeStruct((S, N, 2, P), _f32),
        jax.ShapeDtypeStruct((S, N, 2, P), _f32),
        jax.ShapeDtypeStruct((11, S, P), jnp.bool_),
    ],
)


def kernel(agent_valid, agent_pos, agent_vel, agent_spd, agent_acc,
           agent_yaw_bbox, agent_yaw_rate, agent_type, agent_role,
           agent_size, agent_cmd, map_valid, map_type, map_pos, map_dir,
           tl_valid, tl_state, tl_pos, tl_dir):
    sc_in = (
        agent_valid.transpose(0, 2, 1).astype(_i32).reshape(S, A * T),
        agent_pos.transpose(0, 2, 3, 1).reshape(S, A * 2 * T),
        agent_vel.transpose(0, 2, 3, 1).reshape(S, A * 2 * T),
        agent_spd.transpose(0, 2, 3, 1).reshape(S, A * T),
        agent_acc.transpose(0, 2, 3, 1).reshape(S, A * T),
        agent_yaw_bbox.transpose(0, 2, 3, 1).reshape(S, A * T),
        agent_yaw_rate.transpose(0, 2, 3, 1).reshape(S, A * T),
        agent_type.astype(_i32).reshape(S, A * 3),
        agent_role.astype(_i32).reshape(S, A * 3),
        agent_size.reshape(S, A * 3),
        tl_valid[:, :NH].transpose(0, 2, 1).astype(_i32).reshape(S, TL * NH),
        tl_pos[:, :NH].transpose(0, 3, 2, 1).reshape(S, TL * 2 * NH),
        tl_dir[:, :NH].transpose(0, 3, 2, 1).reshape(S, TL * 2 * NH),
        tl_state[:, :NH].transpose(0, 3, 2, 1).astype(_i32).reshape(
            S, TL * 5 * NH),
    )
    (o_refpos, o_refyaw, o_refrot, o_scval, o_scpos, o_scvel, o_scspd,
     o_scacc, o_scyaw, o_scyrt, o_gtval, o_gtpos, o_gtvel, o_gtspd,
     o_gtyaw, o_typ, o_rol, o_siz, o_tlp, o_tld, o_tls) = _sc_kernel(*sc_in)

    o_mpos, o_mdir, o_mtype = _map_call(
        map_valid.transpose(2, 0, 1),
        map_type.transpose(2, 0, 1),
        map_pos.transpose(0, 2, 3, 1),
        map_dir.transpose(0, 2, 3, 1),
    )

    return (
        o_refpos.transpose(0, 3, 1, 2),
        o_refyaw.transpose(0, 2, 1),
        o_refrot.transpose(0, 3, 1, 2),
        agent_type,
        agent_role,
        o_gtval.astype(bool).reshape(S, A, NF),
        o_gtpos.reshape(S, A, 2, NF).transpose(0, 1, 3, 2),
        o_gtspd.reshape(S, A, NF, 1),
        o_gtvel.reshape(S, A, 2, NF).transpose(0, 1, 3, 2),
        o_gtyaw.reshape(S, A, NF, 1),
        agent_cmd,
        o_scval.astype(bool).reshape(S, A, NH),
        o_scpos.transpose(0, 3, 1, 2),
        o_scvel.transpose(0, 3, 1, 2),
        o_scspd.transpose(0, 3, 1, 2),
        o_scacc.transpose(0, 3, 1, 2),
        o_scyaw.transpose(0, 3, 1, 2),
        o_scyrt.transpose(0, 3, 1, 2),
        o_typ.astype(bool).reshape(S, A, 3),
        o_rol.astype(bool).reshape(S, A, 3),
        o_siz.reshape(S, A, 3),
        map_valid,
        o_mtype.transpose(1, 2, 0),
        o_mpos.transpose(0, 3, 1, 2),
        o_mdir.transpose(0, 3, 1, 2),
        tl_valid[:, :NH],
        o_tls[:, :NH * TL * 5].astype(bool).reshape(S, NH, TL, 5),
        o_tlp.transpose(0, 1, 3, 2),
        o_tld.transpose(0, 1, 3, 2),
    )
